# speedup vs baseline: 1.3969x; 1.3969x over previous
import functools

import jax
import jax.numpy as jnp
from jax import lax
from jax.experimental import pallas as pl
from jax.experimental.pallas import tpu as pltpu
from jax.experimental.pallas import tpu_sc as plsc

NC = 2
NS = 16
NW = NC * NS
LANES = 16


def _make_kernel(B, D, PW, CH):
    n_chunks = PW // CH
    vecs_per_row = D // LANES
    mesh = plsc.VectorSubcoreMesh(
        core_axis_name="c", subcore_axis_name="s",
        num_cores=NC, num_subcores=NS)

    @functools.partial(
        pl.kernel,
        out_type=jax.ShapeDtypeStruct((B, D), jnp.float32),
        mesh=mesh,
        scratch_types=[
            pltpu.VMEM((PW,), jnp.int32),
            pltpu.VMEM((CH, D), jnp.float32),
            pltpu.VMEM((CH, D), jnp.float32),
            pltpu.SemaphoreType.DMA,
            pltpu.SemaphoreType.DMA,
        ],
    )
    def body(mem_hbm, nf_hbm, idx_hbm, out_hbm, idx_v, buf_a, buf_b,
             sem_a, sem_b):
        wid = lax.axis_index("s") * NC + lax.axis_index("c")
        base = jnp.minimum(wid * PW, B - PW)
        pltpu.sync_copy(idx_hbm.at[pl.ds(base, PW)], idx_v)

        def chunk(i, carry):
            ia = idx_v.at[pl.ds(i * CH, CH)]
            ca = pltpu.async_copy(nf_hbm.at[ia], buf_a, sem_a)
            cb = pltpu.async_copy(mem_hbm.at[ia], buf_b, sem_b)
            ca.wait()
            cb.wait()

            def add_row(r, c2):
                for v in range(vecs_per_row):
                    sl = pl.ds(v * LANES, LANES)
                    buf_a[r, sl] = buf_a[r, sl] + buf_b[r, sl]
                return c2

            lax.fori_loop(0, CH, add_row, 0)
            pltpu.sync_copy(buf_a, out_hbm.at[pl.ds(base + i * CH, CH)])
            return carry

        lax.fori_loop(0, n_chunks, chunk, 0)

    return body


def kernel(memory, source_nodes, timestamps, n_layers, node_features):
    del timestamps, n_layers
    B = source_nodes.shape[0]
    D = memory.shape[1]
    CH = 128
    PW = -(-B // NW)
    PW = -(-PW // CH) * CH
    k = _make_kernel(B, D, PW, CH)
    return k(memory, node_features, source_nodes)


# speedup vs baseline: 2.1370x; 1.5299x over previous
import functools

import jax
import jax.numpy as jnp
from jax import lax
from jax.experimental import pallas as pl
from jax.experimental.pallas import tpu as pltpu
from jax.experimental.pallas import tpu_sc as plsc

NC = 2
NS = 16
NW = NC * NS
LANES = 16


def _make_kernel(B, D, PW, CH):
    n_chunks = PW // CH
    vecs_per_row = D // LANES
    mesh = plsc.VectorSubcoreMesh(
        core_axis_name="c", subcore_axis_name="s",
        num_cores=NC, num_subcores=NS)

    @functools.partial(
        pl.kernel,
        out_type=jax.ShapeDtypeStruct((B, D), jnp.float32),
        mesh=mesh,
        scratch_types=[
            pltpu.VMEM((PW,), jnp.int32),
            pltpu.VMEM((CH, D), jnp.float32),
            pltpu.VMEM((CH, D), jnp.float32),
            pltpu.VMEM((CH, D), jnp.float32),
            pltpu.VMEM((CH, D), jnp.float32),
            pltpu.VMEM((CH, D), jnp.float32),
            pltpu.VMEM((CH, D), jnp.float32),
            pltpu.SemaphoreType.DMA,
            pltpu.SemaphoreType.DMA,
            pltpu.SemaphoreType.DMA,
            pltpu.SemaphoreType.DMA,
            pltpu.SemaphoreType.DMA,
            pltpu.SemaphoreType.DMA,
        ],
    )
    def body(mem_hbm, nf_hbm, idx_hbm, out_hbm,
             idx_v, a0, b0, o0, a1, b1, o1,
             sga0, sgb0, sga1, sgb1, sw0, sw1):
        bufs_a = (a0, a1)
        bufs_b = (b0, b1)
        bufs_o = (o0, o1)
        sems_a = (sga0, sga1)
        sems_b = (sgb0, sgb1)
        sems_w = (sw0, sw1)

        wid = lax.axis_index("s") * NC + lax.axis_index("c")
        base = jnp.minimum(wid * PW, B - PW)
        pltpu.sync_copy(idx_hbm.at[pl.ds(base, PW)], idx_v)

        def start_gather(i, slot):
            ia = idx_v.at[pl.ds(i * CH, CH)]
            pltpu.async_copy(nf_hbm.at[ia], bufs_a[slot], sems_a[slot])
            pltpu.async_copy(mem_hbm.at[ia], bufs_b[slot], sems_b[slot])

        start_gather(0, 0)
        start_gather(1, 1)

        def step(g, carry):
            for slot in range(2):
                i = g * 2 + slot
                ia = idx_v.at[pl.ds(i * CH, CH)]
                pltpu.make_async_copy(
                    nf_hbm.at[ia], bufs_a[slot], sems_a[slot]).wait()
                pltpu.make_async_copy(
                    mem_hbm.at[ia], bufs_b[slot], sems_b[slot]).wait()

                @pl.when(i >= 2)
                def _():
                    pltpu.make_async_copy(
                        bufs_o[slot],
                        out_hbm.at[pl.ds(base + (i - 2) * CH, CH)],
                        sems_w[slot]).wait()

                def add_row(r, c2):
                    for v in range(vecs_per_row):
                        sl = pl.ds(v * LANES, LANES)
                        bufs_o[slot][r, sl] = (
                            bufs_a[slot][r, sl] + bufs_b[slot][r, sl])
                    return c2

                lax.fori_loop(0, CH, add_row, 0)

                @pl.when(i + 2 < n_chunks)
                def _():
                    start_gather(i + 2, slot)

                pltpu.async_copy(
                    bufs_o[slot],
                    out_hbm.at[pl.ds(base + i * CH, CH)],
                    sems_w[slot])
            return carry

        lax.fori_loop(0, n_chunks // 2, step, 0)

        for slot in range(2):
            i = n_chunks - 2 + slot
            pltpu.make_async_copy(
                bufs_o[slot],
                out_hbm.at[pl.ds(base + i * CH, CH)],
                sems_w[slot]).wait()

    return body


def kernel(memory, source_nodes, timestamps, n_layers, node_features):
    del timestamps, n_layers
    B = source_nodes.shape[0]
    D = memory.shape[1]
    CH = 112
    PW = 3136
    assert NW * PW >= B and PW % CH == 0 and (PW // CH) % 2 == 0
    assert PW % 8 == 0 and (B - PW) % 8 == 0
    k = _make_kernel(B, D, PW, CH)
    return k(memory, node_features, source_nodes)
